# baseline (speedup 1.0000x reference)
import functools

import jax
import jax.numpy as jnp
from jax import lax
from jax.experimental import pallas as pl
from jax.experimental.pallas import tpu as pltpu
from jax.experimental.pallas import tpu_sc as plsc

VOCAB = 50257
D_MODEL = 1024
BATCH = 32
SEQ = 512

LANES = 16
NUM_WORKERS = 32
TOK_PER_W = (BATCH * SEQ) // NUM_WORKERS
CHUNK = 32
NCHUNK = TOK_PER_W // CHUNK
NBUF = 3
DSLICES = D_MODEL // LANES


def _tree_sum(vals):
    while len(vals) > 1:
        nxt = [vals[i] + vals[i + 1] for i in range(0, len(vals) - 1, 2)]
        if len(vals) % 2:
            nxt.append(vals[-1])
        vals = nxt
    return vals[0]


def _body(ids_hbm, table_hbm, tok_hbm, feat_hbm, idx_v, rows_v, acc_v,
          gsem_arr, ssem_arr):
    c = lax.axis_index("c")
    s = lax.axis_index("s")
    wid = s * 2 + c
    base = pl.multiple_of(wid * TOK_PER_W, TOK_PER_W)

    rows = [rows_v.at[pl.ds(b * CHUNK, CHUNK)] for b in range(NBUF)]
    gsem = [gsem_arr.at[b] for b in range(NBUF)]
    ssem = [ssem_arr.at[b] for b in range(NBUF)]

    pltpu.sync_copy(ids_hbm.at[wid], idx_v)

    def gather(g, p):
        return pltpu.async_copy(
            table_hbm.at[idx_v.at[pl.ds(g * CHUNK, CHUNK)]], rows[p], gsem[p]
        )

    def scatter(g, p):
        return pltpu.async_copy(
            rows[p], tok_hbm.at[pl.ds(base + g * CHUNK, CHUNK)], ssem[p]
        )

    pend_g = [gather(g, g) if g < NBUF - 1 else None for g in range(NBUF)]
    pend_s = [None] * NBUF
    for g in range(NCHUNK):
        p = g % NBUF
        pend_g[p].wait()
        pend_s[p] = scatter(g, p)

        if g + NBUF - 1 < NCHUNK:
            b = (g + NBUF - 1) % NBUF
            if pend_s[b] is not None:
                pend_s[b].wait()
                pend_s[b] = None
            pend_g[b] = gather(g + NBUF - 1, b)

        @plsc.parallel_loop(0, DSLICES, unroll=2)
        def _acc(d):
            ds = pl.ds(d * LANES, LANES)
            t = _tree_sum([rows[p][r, ds] for r in range(CHUNK)])
            if g > 0:
                t = t + acc_v[ds]
            if g == NCHUNK - 1:
                t = t * jnp.float32(1.0 / SEQ)
            acc_v[ds] = t

    for b in range(NBUF):
        if pend_s[b] is not None:
            pend_s[b].wait()
    pltpu.sync_copy(acc_v, feat_hbm.at[wid])


@functools.partial(jax.jit, static_argnames=())
def kernel(input_ids, embedding_table):
    mesh = plsc.VectorSubcoreMesh(core_axis_name="c", subcore_axis_name="s")
    tok, feat = pl.kernel(
        _body,
        out_type=(
            jax.ShapeDtypeStruct((BATCH * SEQ, D_MODEL), jnp.float32),
            jax.ShapeDtypeStruct((BATCH, D_MODEL), jnp.float32),
        ),
        mesh=mesh,
        scratch_types=[
            pltpu.VMEM((TOK_PER_W,), jnp.int32),
            pltpu.VMEM((NBUF * CHUNK, D_MODEL), jnp.float32),
            pltpu.VMEM((D_MODEL,), jnp.float32),
            pltpu.SemaphoreType.DMA((NBUF,)),
            pltpu.SemaphoreType.DMA((NBUF,)),
        ],
    )(input_ids.astype(jnp.int32), embedding_table)
    return (tok.reshape(BATCH, SEQ, D_MODEL), feat)


# speedup vs baseline: 1.0664x; 1.0664x over previous
import functools

import jax
import jax.numpy as jnp
from jax import lax
from jax.experimental import pallas as pl
from jax.experimental.pallas import tpu as pltpu
from jax.experimental.pallas import tpu_sc as plsc

VOCAB = 50257
D_MODEL = 1024
BATCH = 32
SEQ = 512

LANES = 16
NUM_WORKERS = 32
TOK_PER_W = (BATCH * SEQ) // NUM_WORKERS
CHUNK = 32
NCHUNK = TOK_PER_W // CHUNK
NBUF = 3
DSLICES = D_MODEL // LANES


def _tree_sum(vals):
    while len(vals) > 1:
        nxt = [vals[i] + vals[i + 1] for i in range(0, len(vals) - 1, 2)]
        if len(vals) % 2:
            nxt.append(vals[-1])
        vals = nxt
    return vals[0]


def _body(ids_hbm, table_hbm, tok_hbm, feat_hbm, idx_v, rows_v, acc_v,
          gsem_arr, ssem_arr):
    c = lax.axis_index("c")
    s = lax.axis_index("s")
    wid = s * 2 + c
    base = pl.multiple_of(wid * TOK_PER_W, TOK_PER_W)

    rows = [rows_v.at[pl.ds(b * CHUNK, CHUNK)] for b in range(NBUF)]
    gsem = [gsem_arr.at[b] for b in range(NBUF)]
    ssem = [ssem_arr.at[b] for b in range(NBUF)]

    pltpu.sync_copy(ids_hbm.at[wid], idx_v)

    def gather(g, p):
        return pltpu.async_copy(
            table_hbm.at[idx_v.at[pl.ds(g * CHUNK, CHUNK)]], rows[p], gsem[p]
        )

    def scatter(g, p):
        return pltpu.async_copy(
            rows[p], tok_hbm.at[pl.ds(base + g * CHUNK, CHUNK)], ssem[p]
        )

    pend_g = [gather(g, g) if g < NBUF - 1 else None for g in range(NBUF)]
    pend_s = [None] * NBUF
    for g in range(NCHUNK):
        p = g % NBUF
        pend_g[p].wait()
        pend_s[p] = scatter(g, p)

        if g + NBUF - 1 < NCHUNK:
            b = (g + NBUF - 1) % NBUF
            if pend_s[b] is not None:
                pend_s[b].wait()
                pend_s[b] = None
            pend_g[b] = gather(g + NBUF - 1, b)

        @plsc.parallel_loop(0, DSLICES)
        def _acc(d):
            ds = pl.ds(d * LANES, LANES)
            t = _tree_sum([rows[p][r, ds] for r in range(CHUNK)])
            if g > 0:
                t = t + acc_v[ds]
            if g == NCHUNK - 1:
                t = t * jnp.float32(1.0 / SEQ)
            acc_v[ds] = t

    for b in range(NBUF):
        if pend_s[b] is not None:
            pend_s[b].wait()
    pltpu.sync_copy(acc_v, feat_hbm.at[wid])


@functools.partial(jax.jit, static_argnames=())
def kernel(input_ids, embedding_table):
    mesh = plsc.VectorSubcoreMesh(core_axis_name="c", subcore_axis_name="s")
    tok, feat = pl.kernel(
        _body,
        out_type=(
            jax.ShapeDtypeStruct((BATCH * SEQ, D_MODEL), jnp.float32),
            jax.ShapeDtypeStruct((BATCH, D_MODEL), jnp.float32),
        ),
        mesh=mesh,
        scratch_types=[
            pltpu.VMEM((TOK_PER_W,), jnp.int32),
            pltpu.VMEM((NBUF * CHUNK, D_MODEL), jnp.float32),
            pltpu.VMEM((D_MODEL,), jnp.float32),
            pltpu.SemaphoreType.DMA((NBUF,)),
            pltpu.SemaphoreType.DMA((NBUF,)),
        ],
    )(input_ids.astype(jnp.int32), embedding_table)
    return (tok.reshape(BATCH, SEQ, D_MODEL), feat)
